# Initial kernel scaffold; baseline (speedup 1.0000x reference)
import jax
import jax.numpy as jnp
from jax.experimental import pallas as pl


def kernel(X, W1, W2, adj_vals, edge_index):
    raise NotImplementedError("write your pallas kernel here")


# speedup vs baseline: 4.1403x; 4.1403x over previous
import functools

import jax
import jax.numpy as jnp
from jax import lax
from jax.experimental import pallas as pl
from jax.experimental.pallas import tpu as pltpu
from jax.experimental.pallas import tpu_sc as plsc

_N = 10000
_E = 320000
_NX = 128
_NH = 64
_NZ = 16

_POS_W = float(_N * _N - _E) / float(_E)
_NORM = float(_N * _N) / float((_N * _N - _E) * 2)

_NC = 2
_NS = 16
_NW = _NC * _NS
_EW = _E // _NW
_CH = 80
_NCHUNK = _EW // _CH
_RB = 624
_RTAIL = _N - _NS * _RB


def _mm_body(x_ref, w_ref, o_ref):
    o_ref[...] = jnp.dot(x_ref[...], w_ref[...],
                         preferred_element_type=jnp.float32)


def _matmul(x, w, rb=400):
    n, k = x.shape
    m = w.shape[1]
    return pl.pallas_call(
        _mm_body,
        grid=(n // rb,),
        in_specs=[pl.BlockSpec((rb, k), lambda i: (i, 0)),
                  pl.BlockSpec((k, m), lambda i: (0, 0))],
        out_specs=pl.BlockSpec((rb, m), lambda i: (i, 0)),
        out_shape=jax.ShapeDtypeStruct((n, m), jnp.float32),
    )(x, w)


def _layer2_body(p_ref, w_ref, o_ref):
    h = jax.nn.relu(p_ref[0] + p_ref[1])
    o_ref[...] = jnp.dot(h, w_ref[...], preferred_element_type=jnp.float32)


def _layer2(p, w, rb=400):
    _, n, k = p.shape
    m = w.shape[1]
    return pl.pallas_call(
        _layer2_body,
        grid=(n // rb,),
        in_specs=[pl.BlockSpec((2, rb, k), lambda i: (0, i, 0)),
                  pl.BlockSpec((k, m), lambda i: (0, 0))],
        out_specs=pl.BlockSpec((rb, m), lambda i: (i, 0)),
        out_shape=jax.ShapeDtypeStruct((n, m), jnp.float32),
    )(p, w)


def _addp_body(p_ref, o_ref):
    o_ref[...] = p_ref[0] + p_ref[1]


def _add_partials(p, rb=400):
    _, n, k = p.shape
    return pl.pallas_call(
        _addp_body,
        grid=(n // rb,),
        in_specs=[pl.BlockSpec((2, rb, k), lambda i: (0, i, 0))],
        out_specs=pl.BlockSpec((rb, k), lambda i: (i, 0)),
        out_shape=jax.ShapeDtypeStruct((n, k), jnp.float32),
    )(p)


def _make_spmm(d):
    mesh = plsc.VectorSubcoreMesh(core_axis_name="c", subcore_axis_name="s")

    @functools.partial(
        pl.kernel,
        out_type=jax.ShapeDtypeStruct((2, _N, d), jnp.float32),
        mesh=mesh,
        compiler_params=pltpu.CompilerParams(use_tc_tiling_on_sc=False),
        scratch_types=[
            pltpu.VMEM((_CH,), jnp.int32),
            pltpu.VMEM((_CH,), jnp.int32),
            pltpu.VMEM((_CH,), jnp.float32),
            pltpu.VMEM((_CH, d), jnp.float32),
            pltpu.VMEM((_RB, d), jnp.float32),
            pltpu.VMEM_SHARED((_N, d), jnp.float32),
            pltpu.SemaphoreType.DMA,
        ],
    )
    def spmm(table_h, row_h, col_h, val_h, out_h,
             ridx_v, cidx_v, val_v, rows_v, zer_v, acc_sh, sem):
        c = lax.axis_index("c")
        s = lax.axis_index("s")
        wid = s * _NC + c

        def zrow(r, carry):
            for kk in range(d // 16):
                zer_v[r, pl.ds(kk * 16, 16)] = jnp.zeros((16,), jnp.float32)
            return carry

        lax.fori_loop(0, _RB, zrow, 0)
        pltpu.sync_copy(zer_v, acc_sh.at[pl.ds(s * _RB, _RB)])

        @pl.when(s == _NS - 1)
        def _ztail():
            pltpu.sync_copy(zer_v.at[pl.ds(0, _RTAIL)],
                            acc_sh.at[pl.ds(_NS * _RB, _RTAIL)])

        plsc.subcore_barrier()

        base0 = wid * _EW

        def chunk(j, carry):
            base = base0 + j * _CH
            pltpu.sync_copy(row_h.at[pl.ds(base, _CH)], ridx_v)
            pltpu.sync_copy(col_h.at[pl.ds(base, _CH)], cidx_v)
            pltpu.sync_copy(val_h.at[pl.ds(base, _CH)], val_v)
            pltpu.async_copy(table_h.at[cidx_v], rows_v, sem).wait()

            def scale(g, cc):
                vals16 = val_v[pl.ds(g * 16, 16)]
                for t in range(16):
                    v = vals16[t]
                    e = g * 16 + t
                    for kk in range(d // 16):
                        sl = pl.ds(kk * 16, 16)
                        rows_v[e, sl] = rows_v[e, sl] * v
                return cc

            lax.fori_loop(0, _CH // 16, scale, 0)
            pltpu.sync_copy(rows_v, acc_sh.at[ridx_v], add=True)
            return carry

        lax.fori_loop(0, _NCHUNK, chunk, 0)
        plsc.subcore_barrier()
        pltpu.sync_copy(acc_sh.at[pl.ds(s * _RB, _RB)],
                        out_h.at[c, pl.ds(s * _RB, _RB)])

        @pl.when(s == _NS - 1)
        def _wtail():
            pltpu.sync_copy(acc_sh.at[pl.ds(_NS * _RB, _RTAIL)],
                            out_h.at[c, pl.ds(_NS * _RB, _RTAIL)])

    return spmm


def _make_edge_logits():
    mesh = plsc.VectorSubcoreMesh(core_axis_name="c", subcore_axis_name="s")

    @functools.partial(
        pl.kernel,
        out_type=jax.ShapeDtypeStruct((_E,), jnp.float32),
        mesh=mesh,
        compiler_params=pltpu.CompilerParams(use_tc_tiling_on_sc=False,
                                             needs_layout_passes=False),
        scratch_types=[
            pltpu.VMEM((_CH,), jnp.int32),
            pltpu.VMEM((_CH,), jnp.int32),
            pltpu.VMEM((_CH, _NZ), jnp.float32),
            pltpu.VMEM((_CH, _NZ), jnp.float32),
            pltpu.VMEM((_CH,), jnp.float32),
            pltpu.SemaphoreType.DMA,
        ],
    )
    def elog(z_h, row_h, col_h, out_h, ridx_v, cidx_v, zr_v, zc_v, dot_v, sem):
        c = lax.axis_index("c")
        s = lax.axis_index("s")
        wid = s * _NC + c
        base0 = wid * _EW

        def chunk(j, carry):
            base = base0 + j * _CH
            pltpu.sync_copy(row_h.at[pl.ds(base, _CH)], ridx_v)
            pltpu.sync_copy(col_h.at[pl.ds(base, _CH)], cidx_v)
            pltpu.async_copy(z_h.at[ridx_v], zr_v, sem).wait()
            pltpu.async_copy(z_h.at[cidx_v], zc_v, sem).wait()

            def dot_group(g, cc):
                eidx = g * 16 + lax.iota(jnp.int32, 16)
                acc = jnp.zeros((16,), jnp.float32)
                for k in range(_NZ):
                    kidx = jnp.full((16,), k, jnp.int32)
                    acc = acc + (plsc.load_gather(zr_v, [eidx, kidx])
                                 * plsc.load_gather(zc_v, [eidx, kidx]))
                dot_v[pl.ds(g * 16, 16)] = acc
                return cc

            lax.fori_loop(0, _CH // 16, dot_group, 0)
            pltpu.sync_copy(dot_v, out_h.at[pl.ds(base, _CH)])
            return carry

        lax.fori_loop(0, _NCHUNK, chunk, 0)

    return elog


_BM = 512
_G = (_N + _BM - 1) // _BM


def _dec_body(zr_ref, zc_ref, a_ref, s_ref, acc_ref):
    i = pl.program_id(0)
    j = pl.program_id(1)

    @pl.when((i == 0) & (j == 0))
    def _init():
        acc_ref[0] = 0.0

    a = lax.dot_general(zr_ref[...], zc_ref[...],
                        (((1,), (1,)), ((), ())),
                        preferred_element_type=jnp.float32)
    a_ref[...] = a
    sp = jnp.log1p(jnp.exp(-jnp.abs(a))) + jnp.maximum(a, 0.0)
    rows = lax.broadcasted_iota(jnp.int32, (_BM, _BM), 0)
    cols = lax.broadcasted_iota(jnp.int32, (_BM, _BM), 1)
    mask = (rows < _N - i * _BM) & (cols < _N - j * _BM)
    acc_ref[0] += jnp.sum(jnp.where(mask, sp, 0.0))

    @pl.when((i == _G - 1) & (j == _G - 1))
    def _fin():
        s_ref[0, 0] = acc_ref[0]


def _decoder(z):
    return pl.pallas_call(
        _dec_body,
        grid=(_G, _G),
        in_specs=[pl.BlockSpec((_BM, _NZ), lambda i, j: (i, 0)),
                  pl.BlockSpec((_BM, _NZ), lambda i, j: (j, 0))],
        out_specs=[pl.BlockSpec((_BM, _BM), lambda i, j: (i, j)),
                   pl.BlockSpec((1, 1), lambda i, j: (0, 0),
                                memory_space=pltpu.SMEM)],
        out_shape=[jax.ShapeDtypeStruct((_N, _N), jnp.float32),
                   jax.ShapeDtypeStruct((1, 1), jnp.float32)],
        scratch_shapes=[pltpu.SMEM((1,), jnp.float32)],
    )(z, z)


def _loss_body(s_ref, x_ref, o_ref):
    x = x_ref[...]
    lp = jnp.log1p(jnp.exp(-jnp.abs(x)))
    corr = _POS_W * (lp + jnp.maximum(-x, 0.0)) - (lp + jnp.maximum(x, 0.0))
    total = s_ref[0, 0] + jnp.sum(corr)
    o_ref[0, 0] = _NORM * (total / float(_N * _N))


def _loss_finish(s, elog2d):
    return pl.pallas_call(
        _loss_body,
        in_specs=[pl.BlockSpec(memory_space=pltpu.SMEM),
                  pl.BlockSpec((_E // 128, 128), lambda: (0, 0))],
        out_specs=pl.BlockSpec(memory_space=pltpu.SMEM),
        out_shape=jax.ShapeDtypeStruct((1, 1), jnp.float32),
    )(s, elog2d)


def kernel(X, W1, W2, adj_vals, edge_index):
    row = edge_index[0]
    col = edge_index[1]

    xw1 = _matmul(X, W1)
    p1 = _make_spmm(_NH)(xw1, row, col, adj_vals)
    hw2 = _layer2(p1, W2)
    p2 = _make_spmm(_NZ)(hw2, row, col, adj_vals)
    z = _add_partials(p2)
    elog = _make_edge_logits()(z, row, col)
    a, s = _decoder(z)
    loss2d = _loss_finish(s, elog.reshape(_E // 128, 128))
    return (a, loss2d[0, 0])


# speedup vs baseline: 5.6914x; 1.3746x over previous
import functools

import jax
import jax.numpy as jnp
from jax import lax
from jax.experimental import pallas as pl
from jax.experimental.pallas import tpu as pltpu
from jax.experimental.pallas import tpu_sc as plsc

_N = 10000
_E = 320000
_NX = 128
_NH = 64
_NZ = 16

_POS_W = float(_N * _N - _E) / float(_E)
_NORM = float(_N * _N) / float((_N * _N - _E) * 2)

_NC = 2
_NS = 16
_NW = _NC * _NS
_EW = _E // _NW
_CH = 80
_NCHUNK = _EW // _CH
_RB = 624
_RTAIL = _N - _NS * _RB


def _mm_body(x_ref, w_ref, o_ref):
    o_ref[...] = jnp.dot(x_ref[...], w_ref[...],
                         preferred_element_type=jnp.float32)


def _matmul(x, w, rb=400):
    n, k = x.shape
    m = w.shape[1]
    return pl.pallas_call(
        _mm_body,
        grid=(n // rb,),
        in_specs=[pl.BlockSpec((rb, k), lambda i: (i, 0)),
                  pl.BlockSpec((k, m), lambda i: (0, 0))],
        out_specs=pl.BlockSpec((rb, m), lambda i: (i, 0)),
        out_shape=jax.ShapeDtypeStruct((n, m), jnp.float32),
    )(x, w)


def _layer2_body(p_ref, w_ref, o_ref):
    h = jax.nn.relu(p_ref[0] + p_ref[1])
    o_ref[...] = jnp.dot(h, w_ref[...], preferred_element_type=jnp.float32)


def _layer2(p, w, rb=400):
    _, n, k = p.shape
    m = w.shape[1]
    return pl.pallas_call(
        _layer2_body,
        grid=(n // rb,),
        in_specs=[pl.BlockSpec((2, rb, k), lambda i: (0, i, 0)),
                  pl.BlockSpec((k, m), lambda i: (0, 0))],
        out_specs=pl.BlockSpec((rb, m), lambda i: (i, 0)),
        out_shape=jax.ShapeDtypeStruct((n, m), jnp.float32),
    )(p, w)


def _addp_body(p_ref, o_ref):
    o_ref[...] = p_ref[0] + p_ref[1]


def _add_partials(p, rb=400):
    _, n, k = p.shape
    return pl.pallas_call(
        _addp_body,
        grid=(n // rb,),
        in_specs=[pl.BlockSpec((2, rb, k), lambda i: (0, i, 0))],
        out_specs=pl.BlockSpec((rb, k), lambda i: (i, 0)),
        out_shape=jax.ShapeDtypeStruct((n, k), jnp.float32),
    )(p)


def _make_spmm(d):
    mesh = plsc.VectorSubcoreMesh(core_axis_name="c", subcore_axis_name="s")

    @functools.partial(
        pl.kernel,
        out_type=jax.ShapeDtypeStruct((2, _N, d), jnp.float32),
        mesh=mesh,
        compiler_params=pltpu.CompilerParams(use_tc_tiling_on_sc=False),
        scratch_types=[
            pltpu.VMEM((_NCHUNK, _CH), jnp.int32),
            pltpu.VMEM((_NCHUNK, _CH), jnp.int32),
            pltpu.VMEM((_NCHUNK, _CH), jnp.float32),
            pltpu.VMEM((_CH, d), jnp.float32),
            pltpu.VMEM((_CH, d), jnp.float32),
            pltpu.VMEM((_RB, d), jnp.float32),
            pltpu.VMEM_SHARED((_N, d), jnp.float32),
            pltpu.SemaphoreType.DMA,
            pltpu.SemaphoreType.DMA,
            pltpu.SemaphoreType.DMA,
            pltpu.SemaphoreType.DMA,
        ],
    )
    def spmm(table_h, row_h, col_h, val_h, out_h,
             ridx_a, cidx_a, val_a, rows0, rows1, zer_v, acc_sh,
             semg0, semg1, sems0, sems1):
        c = lax.axis_index("c")
        s = lax.axis_index("s")
        wid = s * _NC + c
        rows = (rows0, rows1)
        semg = (semg0, semg1)
        sems = (sems0, sems1)

        def zrow(r, carry):
            for kk in range(d // 16):
                zer_v[r, pl.ds(kk * 16, 16)] = jnp.zeros((16,), jnp.float32)
            return carry

        lax.fori_loop(0, _RB, zrow, 0)
        pltpu.sync_copy(zer_v, acc_sh.at[pl.ds(s * _RB, _RB)])

        @pl.when(s == _NS - 1)
        def _ztail():
            pltpu.sync_copy(zer_v.at[pl.ds(0, _RTAIL)],
                            acc_sh.at[pl.ds(_NS * _RB, _RTAIL)])

        pltpu.sync_copy(row_h.at[wid], ridx_a)
        pltpu.sync_copy(col_h.at[wid], cidx_a)
        pltpu.sync_copy(val_h.at[wid], val_a)
        plsc.subcore_barrier()

        def issue_gather(j, b):
            pltpu.async_copy(table_h.at[cidx_a.at[j]], rows[b], semg[b])

        def wait_gather(b):
            pltpu.make_async_copy(table_h.at[pl.ds(0, _CH)], rows[b],
                                  semg[b]).wait()

        def issue_scatter(j, b):
            pltpu.async_copy(rows[b], acc_sh.at[ridx_a.at[j]], sems[b],
                             add=True)

        def wait_scatter(b):
            pltpu.make_async_copy(table_h.at[pl.ds(0, _CH)], rows[b],
                                  sems[b]).wait()

        def scale(j, b):
            def grp(g, cc):
                vals16 = val_a[j, pl.ds(g * 16, 16)]
                for t in range(16):
                    v = vals16[t]
                    e = g * 16 + t
                    for kk in range(d // 16):
                        sl = pl.ds(kk * 16, 16)
                        rows[b][e, sl] = rows[b][e, sl] * v
                return cc

            lax.fori_loop(0, _CH // 16, grp, 0)

        issue_gather(0, 0)
        wait_gather(0)
        scale(0, 0)
        issue_gather(1, 1)
        issue_scatter(0, 0)

        def pair(m, carry):
            j1 = 2 * m + 1
            wait_gather(1)
            scale(j1, 1)
            wait_scatter(0)
            issue_gather(j1 + 1, 0)
            issue_scatter(j1, 1)
            j2 = 2 * m + 2
            wait_gather(0)
            scale(j2, 0)
            wait_scatter(1)

            @pl.when(j2 + 1 < _NCHUNK)
            def _next():
                issue_gather(j2 + 1, 1)

            issue_scatter(j2, 0)
            return carry

        lax.fori_loop(0, (_NCHUNK - 1) // 2, pair, 0)
        wait_scatter(0)
        plsc.subcore_barrier()
        pltpu.sync_copy(acc_sh.at[pl.ds(s * _RB, _RB)],
                        out_h.at[c, pl.ds(s * _RB, _RB)])

        @pl.when(s == _NS - 1)
        def _wtail():
            pltpu.sync_copy(acc_sh.at[pl.ds(_NS * _RB, _RTAIL)],
                            out_h.at[c, pl.ds(_NS * _RB, _RTAIL)])

    return spmm


def _make_edge_logits():
    mesh = plsc.VectorSubcoreMesh(core_axis_name="c", subcore_axis_name="s")

    @functools.partial(
        pl.kernel,
        out_type=jax.ShapeDtypeStruct((_NW, _NCHUNK, _CH), jnp.float32),
        mesh=mesh,
        compiler_params=pltpu.CompilerParams(use_tc_tiling_on_sc=False,
                                             needs_layout_passes=False),
        scratch_types=[
            pltpu.VMEM((_NCHUNK, _CH), jnp.int32),
            pltpu.VMEM((_NCHUNK, _CH), jnp.int32),
            pltpu.VMEM((_CH, _NZ), jnp.float32),
            pltpu.VMEM((_CH, _NZ), jnp.float32),
            pltpu.VMEM((_CH, _NZ), jnp.float32),
            pltpu.VMEM((_CH, _NZ), jnp.float32),
            pltpu.VMEM((_NCHUNK, _CH), jnp.float32),
            pltpu.SemaphoreType.DMA,
            pltpu.SemaphoreType.DMA,
            pltpu.SemaphoreType.DMA,
            pltpu.SemaphoreType.DMA,
        ],
    )
    def elog(z_h, row_h, col_h, out_h, ridx_a, cidx_a,
             zr0, zr1, zc0, zc1, dot_a, semr0, semr1, semc0, semc1):
        c = lax.axis_index("c")
        s = lax.axis_index("s")
        wid = s * _NC + c
        zr = (zr0, zr1)
        zc = (zc0, zc1)
        semr = (semr0, semr1)
        semc = (semc0, semc1)

        pltpu.sync_copy(row_h.at[wid], ridx_a)
        pltpu.sync_copy(col_h.at[wid], cidx_a)

        def issue(j, b):
            pltpu.async_copy(z_h.at[ridx_a.at[j]], zr[b], semr[b])
            pltpu.async_copy(z_h.at[cidx_a.at[j]], zc[b], semc[b])

        def wait(b):
            pltpu.make_async_copy(z_h.at[pl.ds(0, _CH)], zr[b],
                                  semr[b]).wait()
            pltpu.make_async_copy(z_h.at[pl.ds(0, _CH)], zc[b],
                                  semc[b]).wait()

        def dots(j, b):
            def grp(g, cc):
                eidx = g * 16 + lax.iota(jnp.int32, 16)
                acc = jnp.zeros((16,), jnp.float32)
                for k in range(_NZ):
                    kidx = jnp.full((16,), k, jnp.int32)
                    acc = acc + (plsc.load_gather(zr[b], [eidx, kidx])
                                 * plsc.load_gather(zc[b], [eidx, kidx]))
                dot_a[j, pl.ds(g * 16, 16)] = acc
                return cc

            lax.fori_loop(0, _CH // 16, grp, 0)

        issue(0, 0)

        def pair(m, carry):
            j0 = 2 * m
            j1 = 2 * m + 1
            wait(0)
            issue(j1, 1)
            dots(j0, 0)
            wait(1)

            @pl.when(j1 + 1 < _NCHUNK)
            def _next():
                issue(j1 + 1, 0)

            dots(j1, 1)
            return carry

        lax.fori_loop(0, _NCHUNK // 2, pair, 0)
        wait(0)
        dots(_NCHUNK - 1, 0)
        pltpu.sync_copy(dot_a, out_h.at[wid])

    return elog


_BM = 512
_G = (_N + _BM - 1) // _BM


def _dec_body(zr_ref, zc_ref, a_ref, s_ref, acc_ref):
    i = pl.program_id(0)
    j = pl.program_id(1)

    @pl.when((i == 0) & (j == 0))
    def _init():
        acc_ref[0] = 0.0

    a = lax.dot_general(zr_ref[...], zc_ref[...],
                        (((1,), (1,)), ((), ())),
                        preferred_element_type=jnp.float32)
    a_ref[...] = a
    sp = jnp.log1p(jnp.exp(-jnp.abs(a))) + jnp.maximum(a, 0.0)
    rows = lax.broadcasted_iota(jnp.int32, (_BM, _BM), 0)
    cols = lax.broadcasted_iota(jnp.int32, (_BM, _BM), 1)
    mask = (rows < _N - i * _BM) & (cols < _N - j * _BM)
    acc_ref[0] += jnp.sum(jnp.where(mask, sp, 0.0))

    @pl.when((i == _G - 1) & (j == _G - 1))
    def _fin():
        s_ref[0, 0] = acc_ref[0]


def _decoder(z):
    return pl.pallas_call(
        _dec_body,
        grid=(_G, _G),
        in_specs=[pl.BlockSpec((_BM, _NZ), lambda i, j: (i, 0)),
                  pl.BlockSpec((_BM, _NZ), lambda i, j: (j, 0))],
        out_specs=[pl.BlockSpec((_BM, _BM), lambda i, j: (i, j)),
                   pl.BlockSpec((1, 1), lambda i, j: (0, 0),
                                memory_space=pltpu.SMEM)],
        out_shape=[jax.ShapeDtypeStruct((_N, _N), jnp.float32),
                   jax.ShapeDtypeStruct((1, 1), jnp.float32)],
        scratch_shapes=[pltpu.SMEM((1,), jnp.float32)],
    )(z, z)


def _loss_body(s_ref, x_ref, o_ref):
    x = x_ref[...]
    lp = jnp.log1p(jnp.exp(-jnp.abs(x)))
    corr = _POS_W * (lp + jnp.maximum(-x, 0.0)) - (lp + jnp.maximum(x, 0.0))
    total = s_ref[0, 0] + jnp.sum(corr)
    o_ref[0, 0] = _NORM * (total / float(_N * _N))


def _loss_finish(s, elog2d):
    return pl.pallas_call(
        _loss_body,
        in_specs=[pl.BlockSpec(memory_space=pltpu.SMEM),
                  pl.BlockSpec((_E // 128, 128), lambda: (0, 0))],
        out_specs=pl.BlockSpec(memory_space=pltpu.SMEM),
        out_shape=jax.ShapeDtypeStruct((1, 1), jnp.float32),
    )(s, elog2d)


def kernel(X, W1, W2, adj_vals, edge_index):
    row = edge_index[0].reshape(_NW, _NCHUNK, _CH)
    col = edge_index[1].reshape(_NW, _NCHUNK, _CH)
    val = adj_vals.reshape(_NW, _NCHUNK, _CH)

    xw1 = _matmul(X, W1)
    p1 = _make_spmm(_NH)(xw1, row, col, val)
    hw2 = _layer2(p1, W2)
    p2 = _make_spmm(_NZ)(hw2, row, col, val)
    z = _add_partials(p2)
    elog = _make_edge_logits()(z, row, col)
    a, s = _decoder(z)
    loss2d = _loss_finish(s, elog.reshape(_E // 128, 128))
    return (a, loss2d[0, 0])


# speedup vs baseline: 5.7215x; 1.0053x over previous
import functools

import jax
import jax.numpy as jnp
from jax import lax
from jax.experimental import pallas as pl
from jax.experimental.pallas import tpu as pltpu
from jax.experimental.pallas import tpu_sc as plsc

_N = 10000
_E = 320000
_NX = 128
_NH = 64
_NZ = 16

_POS_W = float(_N * _N - _E) / float(_E)
_NORM = float(_N * _N) / float((_N * _N - _E) * 2)

_NC = 2
_NS = 16
_NW = _NC * _NS
_EW = _E // _NW
_CH = 80
_NCHUNK = _EW // _CH
_RB = 624
_RTAIL = _N - _NS * _RB


def _mm_body(x_ref, w_ref, o_ref):
    o_ref[...] = jnp.dot(x_ref[...], w_ref[...],
                         preferred_element_type=jnp.float32)


def _matmul(x, w, rb=400):
    n, k = x.shape
    m = w.shape[1]
    return pl.pallas_call(
        _mm_body,
        grid=(n // rb,),
        in_specs=[pl.BlockSpec((rb, k), lambda i: (i, 0)),
                  pl.BlockSpec((k, m), lambda i: (0, 0))],
        out_specs=pl.BlockSpec((rb, m), lambda i: (i, 0)),
        out_shape=jax.ShapeDtypeStruct((n, m), jnp.float32),
    )(x, w)


def _layer2_body(p_ref, w_ref, o_ref):
    h = jax.nn.relu(p_ref[0] + p_ref[1])
    o_ref[...] = jnp.dot(h, w_ref[...], preferred_element_type=jnp.float32)


def _layer2(p, w, rb=400):
    _, n, k = p.shape
    m = w.shape[1]
    return pl.pallas_call(
        _layer2_body,
        grid=(n // rb,),
        in_specs=[pl.BlockSpec((2, rb, k), lambda i: (0, i, 0)),
                  pl.BlockSpec((k, m), lambda i: (0, 0))],
        out_specs=pl.BlockSpec((rb, m), lambda i: (i, 0)),
        out_shape=jax.ShapeDtypeStruct((n, m), jnp.float32),
    )(p, w)


def _addp_body(p_ref, o_ref):
    o_ref[...] = p_ref[0] + p_ref[1]


def _add_partials(p, rb=400):
    _, n, k = p.shape
    return pl.pallas_call(
        _addp_body,
        grid=(n // rb,),
        in_specs=[pl.BlockSpec((2, rb, k), lambda i: (0, i, 0))],
        out_specs=pl.BlockSpec((rb, k), lambda i: (i, 0)),
        out_shape=jax.ShapeDtypeStruct((n, k), jnp.float32),
    )(p)


def _make_spmm(d):
    mesh = plsc.VectorSubcoreMesh(core_axis_name="c", subcore_axis_name="s")

    @functools.partial(
        pl.kernel,
        out_type=jax.ShapeDtypeStruct((2, _N, d), jnp.float32),
        mesh=mesh,
        compiler_params=pltpu.CompilerParams(use_tc_tiling_on_sc=False),
        scratch_types=[
            pltpu.VMEM((_NCHUNK, _CH), jnp.int32),
            pltpu.VMEM((_NCHUNK, _CH), jnp.int32),
            pltpu.VMEM((_NCHUNK, _CH), jnp.float32),
            pltpu.VMEM((_CH, d), jnp.float32),
            pltpu.VMEM((_CH, d), jnp.float32),
            pltpu.VMEM((_RB, d), jnp.float32),
            pltpu.VMEM_SHARED((_N, d), jnp.float32),
            pltpu.SemaphoreType.DMA,
            pltpu.SemaphoreType.DMA,
            pltpu.SemaphoreType.DMA,
            pltpu.SemaphoreType.DMA,
        ],
    )
    def spmm(table_h, row_h, col_h, val_h, out_h,
             ridx_a, cidx_a, val_a, rows0, rows1, zer_v, acc_sh,
             semg0, semg1, sems0, sems1):
        c = lax.axis_index("c")
        s = lax.axis_index("s")
        wid = s * _NC + c
        rows = (rows0, rows1)
        semg = (semg0, semg1)
        sems = (sems0, sems1)

        def zrow(r, carry):
            for kk in range(d // 16):
                zer_v[r, pl.ds(kk * 16, 16)] = jnp.zeros((16,), jnp.float32)
            return carry

        lax.fori_loop(0, _RB, zrow, 0)
        pltpu.sync_copy(zer_v, acc_sh.at[pl.ds(s * _RB, _RB)])

        @pl.when(s == _NS - 1)
        def _ztail():
            pltpu.sync_copy(zer_v.at[pl.ds(0, _RTAIL)],
                            acc_sh.at[pl.ds(_NS * _RB, _RTAIL)])

        pltpu.sync_copy(row_h.at[wid], ridx_a)
        pltpu.sync_copy(col_h.at[wid], cidx_a)
        pltpu.sync_copy(val_h.at[wid], val_a)
        plsc.subcore_barrier()

        def issue_gather(j, b):
            pltpu.async_copy(table_h.at[cidx_a.at[j]], rows[b], semg[b])

        def wait_gather(b):
            pltpu.make_async_copy(table_h.at[pl.ds(0, _CH)], rows[b],
                                  semg[b]).wait()

        def issue_scatter(j, b):
            pltpu.async_copy(rows[b], acc_sh.at[ridx_a.at[j]], sems[b],
                             add=True)

        def wait_scatter(b):
            pltpu.make_async_copy(table_h.at[pl.ds(0, _CH)], rows[b],
                                  sems[b]).wait()

        def scale(j, b):
            def grp(g, cc):
                vals16 = val_a[j, pl.ds(g * 16, 16)]
                for t in range(16):
                    v = vals16[t]
                    e = g * 16 + t
                    for kk in range(d // 16):
                        sl = pl.ds(kk * 16, 16)
                        rows[b][e, sl] = rows[b][e, sl] * v
                return cc

            lax.fori_loop(0, _CH // 16, grp, 0)

        issue_gather(0, 0)
        wait_gather(0)
        scale(0, 0)
        issue_gather(1, 1)
        issue_scatter(0, 0)

        def pair(m, carry):
            j1 = 2 * m + 1
            wait_gather(1)
            scale(j1, 1)
            wait_scatter(0)
            issue_gather(j1 + 1, 0)
            issue_scatter(j1, 1)
            j2 = 2 * m + 2
            wait_gather(0)
            scale(j2, 0)
            wait_scatter(1)

            @pl.when(j2 + 1 < _NCHUNK)
            def _next():
                issue_gather(j2 + 1, 1)

            issue_scatter(j2, 0)
            return carry

        lax.fori_loop(0, (_NCHUNK - 1) // 2, pair, 0)
        wait_scatter(0)
        plsc.subcore_barrier()
        pltpu.sync_copy(acc_sh.at[pl.ds(s * _RB, _RB)],
                        out_h.at[c, pl.ds(s * _RB, _RB)])

        @pl.when(s == _NS - 1)
        def _wtail():
            pltpu.sync_copy(acc_sh.at[pl.ds(_NS * _RB, _RTAIL)],
                            out_h.at[c, pl.ds(_NS * _RB, _RTAIL)])

    return spmm


def _make_edge_logits():
    mesh = plsc.VectorSubcoreMesh(core_axis_name="c", subcore_axis_name="s")

    @functools.partial(
        pl.kernel,
        out_type=jax.ShapeDtypeStruct((_NW, _NCHUNK, _CH), jnp.float32),
        mesh=mesh,
        compiler_params=pltpu.CompilerParams(use_tc_tiling_on_sc=False,
                                             needs_layout_passes=False),
        scratch_types=[
            pltpu.VMEM((_NCHUNK, _CH), jnp.int32),
            pltpu.VMEM((_NCHUNK, _CH), jnp.int32),
            pltpu.VMEM((_CH, _NZ), jnp.float32),
            pltpu.VMEM((_CH, _NZ), jnp.float32),
            pltpu.VMEM((_CH, _NZ), jnp.float32),
            pltpu.VMEM((_CH, _NZ), jnp.float32),
            pltpu.VMEM((_NCHUNK, _CH), jnp.float32),
            pltpu.SemaphoreType.DMA,
            pltpu.SemaphoreType.DMA,
            pltpu.SemaphoreType.DMA,
            pltpu.SemaphoreType.DMA,
        ],
    )
    def elog(z_h, row_h, col_h, out_h, ridx_a, cidx_a,
             zr0, zr1, zc0, zc1, dot_a, semr0, semr1, semc0, semc1):
        c = lax.axis_index("c")
        s = lax.axis_index("s")
        wid = s * _NC + c
        zr = (zr0, zr1)
        zc = (zc0, zc1)
        semr = (semr0, semr1)
        semc = (semc0, semc1)

        pltpu.sync_copy(row_h.at[wid], ridx_a)
        pltpu.sync_copy(col_h.at[wid], cidx_a)

        def issue(j, b):
            pltpu.async_copy(z_h.at[ridx_a.at[j]], zr[b], semr[b])
            pltpu.async_copy(z_h.at[cidx_a.at[j]], zc[b], semc[b])

        def wait(b):
            pltpu.make_async_copy(z_h.at[pl.ds(0, _CH)], zr[b],
                                  semr[b]).wait()
            pltpu.make_async_copy(z_h.at[pl.ds(0, _CH)], zc[b],
                                  semc[b]).wait()

        def dots(j, b):
            def grp(g, cc):
                eidx = g * 16 + lax.iota(jnp.int32, 16)
                acc = jnp.zeros((16,), jnp.float32)
                for k in range(_NZ):
                    kidx = jnp.full((16,), k, jnp.int32)
                    acc = acc + (plsc.load_gather(zr[b], [eidx, kidx])
                                 * plsc.load_gather(zc[b], [eidx, kidx]))
                dot_a[j, pl.ds(g * 16, 16)] = acc
                return cc

            lax.fori_loop(0, _CH // 16, grp, 0)

        issue(0, 0)

        def pair(m, carry):
            j0 = 2 * m
            j1 = 2 * m + 1
            wait(0)
            issue(j1, 1)
            dots(j0, 0)
            wait(1)

            @pl.when(j1 + 1 < _NCHUNK)
            def _next():
                issue(j1 + 1, 0)

            dots(j1, 1)
            return carry

        lax.fori_loop(0, _NCHUNK // 2, pair, 0)
        wait(0)
        dots(_NCHUNK - 1, 0)
        pltpu.sync_copy(dot_a, out_h.at[wid])

    return elog


_BM = 512
_G = (_N + _BM - 1) // _BM


def _dec_body(zr_ref, zc_ref, a_ref, s_ref, acc_ref):
    i = pl.program_id(0)
    j = pl.program_id(1)

    @pl.when((i == 0) & (j == 0))
    def _init():
        acc_ref[0] = 0.0

    a = lax.dot_general(zr_ref[...], zc_ref[...],
                        (((1,), (1,)), ((), ())),
                        preferred_element_type=jnp.float32)
    a_ref[...] = a
    sp = jnp.log1p(jnp.exp(-jnp.abs(a))) + jnp.maximum(a, 0.0)
    acc_ref[0] += jnp.sum(sp)

    @pl.when((i == _G - 1) & (j == _G - 1))
    def _fin():
        s_ref[0, 0] = acc_ref[0]


_NPAD = _G * _BM * _G * _BM - _N * _N


def _decoder(zpad):
    return pl.pallas_call(
        _dec_body,
        grid=(_G, _G),
        in_specs=[pl.BlockSpec((_BM, _NZ), lambda i, j: (i, 0)),
                  pl.BlockSpec((_BM, _NZ), lambda i, j: (j, 0))],
        out_specs=[pl.BlockSpec((_BM, _BM), lambda i, j: (i, j)),
                   pl.BlockSpec((1, 1), lambda i, j: (0, 0),
                                memory_space=pltpu.SMEM)],
        out_shape=[jax.ShapeDtypeStruct((_N, _N), jnp.float32),
                   jax.ShapeDtypeStruct((1, 1), jnp.float32)],
        scratch_shapes=[pltpu.SMEM((1,), jnp.float32)],
    )(zpad, zpad)


def _loss_body(s_ref, x_ref, o_ref):
    x = x_ref[...]
    lp = jnp.log1p(jnp.exp(-jnp.abs(x)))
    corr = _POS_W * (lp + jnp.maximum(-x, 0.0)) - (lp + jnp.maximum(x, 0.0))
    zero = jnp.float32(0.0)
    sp0 = jnp.log1p(jnp.exp(-jnp.abs(zero))) + jnp.maximum(zero, 0.0)
    total = s_ref[0, 0] - jnp.float32(_NPAD) * sp0 + jnp.sum(corr)
    o_ref[0, 0] = _NORM * (total / float(_N * _N))


def _loss_finish(s, elog2d):
    return pl.pallas_call(
        _loss_body,
        in_specs=[pl.BlockSpec(memory_space=pltpu.SMEM),
                  pl.BlockSpec((_E // 128, 128), lambda: (0, 0))],
        out_specs=pl.BlockSpec(memory_space=pltpu.SMEM),
        out_shape=jax.ShapeDtypeStruct((1, 1), jnp.float32),
    )(s, elog2d)


def kernel(X, W1, W2, adj_vals, edge_index):
    row = edge_index[0].reshape(_NW, _NCHUNK, _CH)
    col = edge_index[1].reshape(_NW, _NCHUNK, _CH)
    val = adj_vals.reshape(_NW, _NCHUNK, _CH)

    xw1 = _matmul(X, W1)
    p1 = _make_spmm(_NH)(xw1, row, col, val)
    hw2 = _layer2(p1, W2)
    p2 = _make_spmm(_NZ)(hw2, row, col, val)
    z = _add_partials(p2)
    elog = _make_edge_logits()(z, row, col)
    zpad = jnp.pad(z, ((0, _G * _BM - _N), (0, 0)))
    a, s = _decoder(zpad)
    loss2d = _loss_finish(s, elog.reshape(_E // 128, 128))
    return (a, loss2d[0, 0])


# speedup vs baseline: 7.0261x; 1.2280x over previous
import functools

import jax
import jax.numpy as jnp
from jax import lax
from jax.experimental import pallas as pl
from jax.experimental.pallas import tpu as pltpu
from jax.experimental.pallas import tpu_sc as plsc

_N = 10000
_E = 320000
_NX = 128
_NH = 64
_NZ = 16

_POS_W = float(_N * _N - _E) / float(_E)
_NORM = float(_N * _N) / float((_N * _N - _E) * 2)

_NC = 2
_NS = 16
_NW = _NC * _NS
_EW = _E // _NW
_CH = 80
_NCHUNK = _EW // _CH
_RB = 624
_RTAIL = _N - _NS * _RB


def _mm_body(x_ref, w_ref, o_ref):
    o_ref[...] = jnp.dot(x_ref[...], w_ref[...],
                         preferred_element_type=jnp.float32)


def _matmul(x, w, rb=400):
    n, k = x.shape
    m = w.shape[1]
    return pl.pallas_call(
        _mm_body,
        grid=(n // rb,),
        in_specs=[pl.BlockSpec((rb, k), lambda i: (i, 0)),
                  pl.BlockSpec((k, m), lambda i: (0, 0))],
        out_specs=pl.BlockSpec((rb, m), lambda i: (i, 0)),
        out_shape=jax.ShapeDtypeStruct((n, m), jnp.float32),
    )(x, w)


def _layer2_body(p_ref, w_ref, o_ref):
    h = jax.nn.relu(p_ref[0] + p_ref[1])
    o_ref[...] = jnp.dot(h, w_ref[...], preferred_element_type=jnp.float32)


def _layer2(p, w, rb=400):
    _, n, k = p.shape
    m = w.shape[1]
    return pl.pallas_call(
        _layer2_body,
        grid=(n // rb,),
        in_specs=[pl.BlockSpec((2, rb, k), lambda i: (0, i, 0)),
                  pl.BlockSpec((k, m), lambda i: (0, 0))],
        out_specs=pl.BlockSpec((rb, m), lambda i: (i, 0)),
        out_shape=jax.ShapeDtypeStruct((n, m), jnp.float32),
    )(p, w)


def _addp_body(p_ref, o_ref):
    o_ref[...] = p_ref[0] + p_ref[1]


def _add_partials(p, rb=400):
    _, n, k = p.shape
    return pl.pallas_call(
        _addp_body,
        grid=(n // rb,),
        in_specs=[pl.BlockSpec((2, rb, k), lambda i: (0, i, 0))],
        out_specs=pl.BlockSpec((rb, k), lambda i: (i, 0)),
        out_shape=jax.ShapeDtypeStruct((n, k), jnp.float32),
    )(p)


def _make_spmm(d):
    mesh = plsc.VectorSubcoreMesh(core_axis_name="c", subcore_axis_name="s")

    @functools.partial(
        pl.kernel,
        out_type=jax.ShapeDtypeStruct((2, _N, d), jnp.float32),
        mesh=mesh,
        compiler_params=pltpu.CompilerParams(use_tc_tiling_on_sc=False),
        scratch_types=[
            pltpu.VMEM((_NCHUNK, _CH), jnp.int32),
            pltpu.VMEM((_NCHUNK, _CH), jnp.int32),
            pltpu.VMEM((_NCHUNK, _CH), jnp.float32),
            pltpu.VMEM((_CH, d), jnp.float32),
            pltpu.VMEM((_CH, d), jnp.float32),
            pltpu.VMEM((_RB, d), jnp.float32),
            pltpu.VMEM_SHARED((_N, d), jnp.float32),
            pltpu.SemaphoreType.DMA,
            pltpu.SemaphoreType.DMA,
            pltpu.SemaphoreType.DMA,
            pltpu.SemaphoreType.DMA,
        ],
    )
    def spmm(table_h, row_h, col_h, val_h, out_h,
             ridx_a, cidx_a, val_a, rows0, rows1, zer_v, acc_sh,
             semg0, semg1, sems0, sems1):
        c = lax.axis_index("c")
        s = lax.axis_index("s")
        wid = s * _NC + c
        rows = (rows0, rows1)
        semg = (semg0, semg1)
        sems = (sems0, sems1)

        def zrow(r, carry):
            for kk in range(d // 16):
                zer_v[r, pl.ds(kk * 16, 16)] = jnp.zeros((16,), jnp.float32)
            return carry

        lax.fori_loop(0, _RB, zrow, 0)
        pltpu.sync_copy(zer_v, acc_sh.at[pl.ds(s * _RB, _RB)])

        @pl.when(s == _NS - 1)
        def _ztail():
            pltpu.sync_copy(zer_v.at[pl.ds(0, _RTAIL)],
                            acc_sh.at[pl.ds(_NS * _RB, _RTAIL)])

        pltpu.sync_copy(row_h.at[wid], ridx_a)
        pltpu.sync_copy(col_h.at[wid], cidx_a)
        pltpu.sync_copy(val_h.at[wid], val_a)
        plsc.subcore_barrier()

        def issue_gather(j, b):
            pltpu.async_copy(table_h.at[cidx_a.at[j]], rows[b], semg[b])

        def wait_gather(b):
            pltpu.make_async_copy(table_h.at[pl.ds(0, _CH)], rows[b],
                                  semg[b]).wait()

        def issue_scatter(j, b):
            pltpu.async_copy(rows[b], acc_sh.at[ridx_a.at[j]], sems[b],
                             add=True)

        def wait_scatter(b):
            pltpu.make_async_copy(table_h.at[pl.ds(0, _CH)], rows[b],
                                  sems[b]).wait()

        def scale(j, b):
            def grp(g, cc):
                vals16 = val_a[j, pl.ds(g * 16, 16)]
                for t in range(16):
                    v = vals16[t]
                    e = g * 16 + t
                    for kk in range(d // 16):
                        sl = pl.ds(kk * 16, 16)
                        rows[b][e, sl] = rows[b][e, sl] * v
                return cc

            lax.fori_loop(0, _CH // 16, grp, 0)

        issue_gather(0, 0)
        wait_gather(0)
        scale(0, 0)
        issue_gather(1, 1)
        issue_scatter(0, 0)

        def pair(m, carry):
            j1 = 2 * m + 1
            wait_gather(1)
            scale(j1, 1)
            wait_scatter(0)
            issue_gather(j1 + 1, 0)
            issue_scatter(j1, 1)
            j2 = 2 * m + 2
            wait_gather(0)
            scale(j2, 0)
            wait_scatter(1)

            @pl.when(j2 + 1 < _NCHUNK)
            def _next():
                issue_gather(j2 + 1, 1)

            issue_scatter(j2, 0)
            return carry

        lax.fori_loop(0, (_NCHUNK - 1) // 2, pair, 0)
        wait_scatter(0)
        plsc.subcore_barrier()
        pltpu.sync_copy(acc_sh.at[pl.ds(s * _RB, _RB)],
                        out_h.at[c, pl.ds(s * _RB, _RB)])

        @pl.when(s == _NS - 1)
        def _wtail():
            pltpu.sync_copy(acc_sh.at[pl.ds(_NS * _RB, _RTAIL)],
                            out_h.at[c, pl.ds(_NS * _RB, _RTAIL)])

    return spmm


def _make_edge_logits():
    mesh = plsc.VectorSubcoreMesh(core_axis_name="c", subcore_axis_name="s")

    @functools.partial(
        pl.kernel,
        out_type=jax.ShapeDtypeStruct((_NW, _NCHUNK, _CH), jnp.float32),
        mesh=mesh,
        compiler_params=pltpu.CompilerParams(use_tc_tiling_on_sc=False,
                                             needs_layout_passes=False),
        scratch_types=[
            pltpu.VMEM((_NCHUNK, _CH), jnp.int32),
            pltpu.VMEM((_NCHUNK, _CH), jnp.int32),
            pltpu.VMEM((_CH, _NZ), jnp.float32),
            pltpu.VMEM((_CH, _NZ), jnp.float32),
            pltpu.VMEM((_CH, _NZ), jnp.float32),
            pltpu.VMEM((_CH, _NZ), jnp.float32),
            pltpu.VMEM((_NCHUNK, _CH), jnp.float32),
            pltpu.SemaphoreType.DMA,
            pltpu.SemaphoreType.DMA,
            pltpu.SemaphoreType.DMA,
            pltpu.SemaphoreType.DMA,
        ],
    )
    def elog(z_h, row_h, col_h, out_h, ridx_a, cidx_a,
             zr0, zr1, zc0, zc1, dot_a, semr0, semr1, semc0, semc1):
        c = lax.axis_index("c")
        s = lax.axis_index("s")
        wid = s * _NC + c
        zr = (zr0, zr1)
        zc = (zc0, zc1)
        semr = (semr0, semr1)
        semc = (semc0, semc1)

        pltpu.sync_copy(row_h.at[wid], ridx_a)
        pltpu.sync_copy(col_h.at[wid], cidx_a)

        def issue(j, b):
            pltpu.async_copy(z_h.at[ridx_a.at[j]], zr[b], semr[b])
            pltpu.async_copy(z_h.at[cidx_a.at[j]], zc[b], semc[b])

        def wait(b):
            pltpu.make_async_copy(z_h.at[pl.ds(0, _CH)], zr[b],
                                  semr[b]).wait()
            pltpu.make_async_copy(z_h.at[pl.ds(0, _CH)], zc[b],
                                  semc[b]).wait()

        def dots(j, b):
            def grp(g, cc):
                eidx = g * 16 + lax.iota(jnp.int32, 16)
                acc = jnp.zeros((16,), jnp.float32)
                for k in range(_NZ):
                    kidx = jnp.full((16,), k, jnp.int32)
                    acc = acc + (plsc.load_gather(zr[b], [eidx, kidx])
                                 * plsc.load_gather(zc[b], [eidx, kidx]))
                dot_a[j, pl.ds(g * 16, 16)] = acc
                return cc

            lax.fori_loop(0, _CH // 16, grp, 0)

        issue(0, 0)

        def pair(m, carry):
            j0 = 2 * m
            j1 = 2 * m + 1
            wait(0)
            issue(j1, 1)
            dots(j0, 0)
            wait(1)

            @pl.when(j1 + 1 < _NCHUNK)
            def _next():
                issue(j1 + 1, 0)

            dots(j1, 1)
            return carry

        lax.fori_loop(0, _NCHUNK // 2, pair, 0)
        wait(0)
        dots(_NCHUNK - 1, 0)
        pltpu.sync_copy(dot_a, out_h.at[wid])

    return elog


_BM = 1024
_G = (_N + _BM - 1) // _BM


def _dec_body(zr_ref, zc_ref, a_ref, s_ref, acc_ref):
    i = pl.program_id(0)
    j = pl.program_id(1)

    @pl.when((i == 0) & (j == 0))
    def _init():
        acc_ref[0] = 0.0

    a = lax.dot_general(zr_ref[...], zc_ref[...],
                        (((1,), (1,)), ((), ())),
                        preferred_element_type=jnp.float32)
    a_ref[...] = a
    sp = jnp.log1p(jnp.exp(-jnp.abs(a))) + jnp.maximum(a, 0.0)
    acc_ref[0] += jnp.sum(sp)

    @pl.when((i == _G - 1) & (j == _G - 1))
    def _fin():
        s_ref[0, 0] = acc_ref[0]


_NPAD = _G * _BM * _G * _BM - _N * _N


def _decoder(zpad):
    return pl.pallas_call(
        _dec_body,
        grid=(_G, _G),
        in_specs=[pl.BlockSpec((_BM, _NZ), lambda i, j: (i, 0)),
                  pl.BlockSpec((_BM, _NZ), lambda i, j: (j, 0))],
        out_specs=[pl.BlockSpec((_BM, _BM), lambda i, j: (i, j)),
                   pl.BlockSpec((1, 1), lambda i, j: (0, 0),
                                memory_space=pltpu.SMEM)],
        out_shape=[jax.ShapeDtypeStruct((_N, _N), jnp.float32),
                   jax.ShapeDtypeStruct((1, 1), jnp.float32)],
        scratch_shapes=[pltpu.SMEM((1,), jnp.float32)],
    )(zpad, zpad)


def _loss_body(s_ref, x_ref, o_ref):
    x = x_ref[...]
    lp = jnp.log1p(jnp.exp(-jnp.abs(x)))
    corr = _POS_W * (lp + jnp.maximum(-x, 0.0)) - (lp + jnp.maximum(x, 0.0))
    zero = jnp.float32(0.0)
    sp0 = jnp.log1p(jnp.exp(-jnp.abs(zero))) + jnp.maximum(zero, 0.0)
    total = s_ref[0, 0] - jnp.float32(_NPAD) * sp0 + jnp.sum(corr)
    o_ref[0, 0] = _NORM * (total / float(_N * _N))


def _loss_finish(s, elog2d):
    return pl.pallas_call(
        _loss_body,
        in_specs=[pl.BlockSpec(memory_space=pltpu.SMEM),
                  pl.BlockSpec((_E // 128, 128), lambda: (0, 0))],
        out_specs=pl.BlockSpec(memory_space=pltpu.SMEM),
        out_shape=jax.ShapeDtypeStruct((1, 1), jnp.float32),
    )(s, elog2d)


def kernel(X, W1, W2, adj_vals, edge_index):
    row = edge_index[0].reshape(_NW, _NCHUNK, _CH)
    col = edge_index[1].reshape(_NW, _NCHUNK, _CH)
    val = adj_vals.reshape(_NW, _NCHUNK, _CH)

    xw1 = _matmul(X, W1)
    p1 = _make_spmm(_NH)(xw1, row, col, val)
    hw2 = _layer2(p1, W2)
    p2 = _make_spmm(_NZ)(hw2, row, col, val)
    z = _add_partials(p2)
    elog = _make_edge_logits()(z, row, col)
    zpad = jnp.pad(z, ((0, _G * _BM - _N), (0, 0)))
    a, s = _decoder(zpad)
    loss2d = _loss_finish(s, elog.reshape(_E // 128, 128))
    return (a, loss2d[0, 0])


# speedup vs baseline: 7.2125x; 1.0265x over previous
import functools

import jax
import jax.numpy as jnp
from jax import lax
from jax.experimental import pallas as pl
from jax.experimental.pallas import tpu as pltpu
from jax.experimental.pallas import tpu_sc as plsc

_N = 10000
_E = 320000
_NX = 128
_NH = 64
_NZ = 16

_POS_W = float(_N * _N - _E) / float(_E)
_NORM = float(_N * _N) / float((_N * _N - _E) * 2)

_NC = 2
_NS = 16
_NW = _NC * _NS
_EW = _E // _NW
_CH = 80
_NCHUNK = _EW // _CH
_RB = 624
_RTAIL = _N - _NS * _RB


def _mm_body(x_ref, w_ref, o_ref):
    o_ref[...] = jnp.dot(x_ref[...], w_ref[...],
                         preferred_element_type=jnp.float32)


def _matmul(x, w, rb=400):
    n, k = x.shape
    m = w.shape[1]
    return pl.pallas_call(
        _mm_body,
        grid=(n // rb,),
        in_specs=[pl.BlockSpec((rb, k), lambda i: (i, 0)),
                  pl.BlockSpec((k, m), lambda i: (0, 0))],
        out_specs=pl.BlockSpec((rb, m), lambda i: (i, 0)),
        out_shape=jax.ShapeDtypeStruct((n, m), jnp.float32),
    )(x, w)


def _layer2_body(p_ref, w_ref, o_ref):
    h = jax.nn.relu(p_ref[0] + p_ref[1])
    o_ref[...] = jnp.dot(h, w_ref[...], preferred_element_type=jnp.float32)


def _layer2(p, w, rb=400):
    _, n, k = p.shape
    m = w.shape[1]
    return pl.pallas_call(
        _layer2_body,
        grid=(n // rb,),
        in_specs=[pl.BlockSpec((2, rb, k), lambda i: (0, i, 0)),
                  pl.BlockSpec((k, m), lambda i: (0, 0))],
        out_specs=pl.BlockSpec((rb, m), lambda i: (i, 0)),
        out_shape=jax.ShapeDtypeStruct((n, m), jnp.float32),
    )(p, w)


def _addp_body(p_ref, o_ref):
    o_ref[...] = p_ref[0] + p_ref[1]


def _add_partials(p, rb=400):
    _, n, k = p.shape
    return pl.pallas_call(
        _addp_body,
        grid=(n // rb,),
        in_specs=[pl.BlockSpec((2, rb, k), lambda i: (0, i, 0))],
        out_specs=pl.BlockSpec((rb, k), lambda i: (i, 0)),
        out_shape=jax.ShapeDtypeStruct((n, k), jnp.float32),
    )(p)


def _make_spmm(d, n_out=_N):
    mesh = plsc.VectorSubcoreMesh(core_axis_name="c", subcore_axis_name="s")

    @functools.partial(
        pl.kernel,
        out_type=jax.ShapeDtypeStruct((2, n_out, d), jnp.float32),
        mesh=mesh,
        compiler_params=pltpu.CompilerParams(use_tc_tiling_on_sc=False),
        scratch_types=[
            pltpu.VMEM((_NCHUNK, _CH), jnp.int32),
            pltpu.VMEM((_NCHUNK, _CH), jnp.int32),
            pltpu.VMEM((_NCHUNK, _CH), jnp.float32),
            pltpu.VMEM((_CH, d), jnp.float32),
            pltpu.VMEM((_CH, d), jnp.float32),
            pltpu.VMEM((_RB, d), jnp.float32),
            pltpu.VMEM_SHARED((_N, d), jnp.float32),
            pltpu.SemaphoreType.DMA,
            pltpu.SemaphoreType.DMA,
            pltpu.SemaphoreType.DMA,
            pltpu.SemaphoreType.DMA,
        ],
    )
    def spmm(table_h, row_h, col_h, val_h, out_h,
             ridx_a, cidx_a, val_a, rows0, rows1, zer_v, acc_sh,
             semg0, semg1, sems0, sems1):
        c = lax.axis_index("c")
        s = lax.axis_index("s")
        wid = s * _NC + c
        rows = (rows0, rows1)
        semg = (semg0, semg1)
        sems = (sems0, sems1)

        def zrow(r, carry):
            for kk in range(d // 16):
                zer_v[r, pl.ds(kk * 16, 16)] = jnp.zeros((16,), jnp.float32)
            return carry

        lax.fori_loop(0, _RB, zrow, 0)
        pltpu.sync_copy(zer_v, acc_sh.at[pl.ds(s * _RB, _RB)])

        @pl.when(s == _NS - 1)
        def _ztail():
            pltpu.sync_copy(zer_v.at[pl.ds(0, _RTAIL)],
                            acc_sh.at[pl.ds(_NS * _RB, _RTAIL)])

        pltpu.sync_copy(row_h.at[wid], ridx_a)
        pltpu.sync_copy(col_h.at[wid], cidx_a)
        pltpu.sync_copy(val_h.at[wid], val_a)
        plsc.subcore_barrier()

        def issue_gather(j, b):
            pltpu.async_copy(table_h.at[cidx_a.at[j]], rows[b], semg[b])

        def wait_gather(b):
            pltpu.make_async_copy(table_h.at[pl.ds(0, _CH)], rows[b],
                                  semg[b]).wait()

        def issue_scatter(j, b):
            pltpu.async_copy(rows[b], acc_sh.at[ridx_a.at[j]], sems[b],
                             add=True)

        def wait_scatter(b):
            pltpu.make_async_copy(table_h.at[pl.ds(0, _CH)], rows[b],
                                  sems[b]).wait()

        def scale(j, b):
            def grp(g, cc):
                vals16 = val_a[j, pl.ds(g * 16, 16)]
                for t in range(16):
                    v = vals16[t]
                    e = g * 16 + t
                    for kk in range(d // 16):
                        sl = pl.ds(kk * 16, 16)
                        rows[b][e, sl] = rows[b][e, sl] * v
                return cc

            lax.fori_loop(0, _CH // 16, grp, 0)

        issue_gather(0, 0)
        wait_gather(0)
        scale(0, 0)
        issue_gather(1, 1)
        issue_scatter(0, 0)

        def pair(m, carry):
            j1 = 2 * m + 1
            wait_gather(1)
            scale(j1, 1)
            wait_scatter(0)
            issue_gather(j1 + 1, 0)
            issue_scatter(j1, 1)
            j2 = 2 * m + 2
            wait_gather(0)
            scale(j2, 0)
            wait_scatter(1)

            @pl.when(j2 + 1 < _NCHUNK)
            def _next():
                issue_gather(j2 + 1, 1)

            issue_scatter(j2, 0)
            return carry

        lax.fori_loop(0, (_NCHUNK - 1) // 2, pair, 0)
        wait_scatter(0)
        plsc.subcore_barrier()
        pltpu.sync_copy(acc_sh.at[pl.ds(s * _RB, _RB)],
                        out_h.at[c, pl.ds(s * _RB, _RB)])

        @pl.when(s == _NS - 1)
        def _wtail():
            pltpu.sync_copy(acc_sh.at[pl.ds(_NS * _RB, _RTAIL)],
                            out_h.at[c, pl.ds(_NS * _RB, _RTAIL)])

        if n_out > _N:
            @pl.when(s == 0)
            def _ptail():
                pltpu.sync_copy(zer_v.at[pl.ds(0, n_out - _N)],
                                out_h.at[c, pl.ds(_N, n_out - _N)])

    return spmm


def _make_edge_logits():
    mesh = plsc.VectorSubcoreMesh(core_axis_name="c", subcore_axis_name="s")

    @functools.partial(
        pl.kernel,
        out_type=jax.ShapeDtypeStruct((_NW, _NCHUNK, _CH), jnp.float32),
        mesh=mesh,
        compiler_params=pltpu.CompilerParams(use_tc_tiling_on_sc=False,
                                             needs_layout_passes=False),
        scratch_types=[
            pltpu.VMEM((_NCHUNK, _CH), jnp.int32),
            pltpu.VMEM((_NCHUNK, _CH), jnp.int32),
            pltpu.VMEM((_CH, _NZ), jnp.float32),
            pltpu.VMEM((_CH, _NZ), jnp.float32),
            pltpu.VMEM((_CH, _NZ), jnp.float32),
            pltpu.VMEM((_CH, _NZ), jnp.float32),
            pltpu.VMEM((_CH, _NZ), jnp.float32),
            pltpu.VMEM((_CH, _NZ), jnp.float32),
            pltpu.VMEM((_CH, _NZ), jnp.float32),
            pltpu.VMEM((_CH, _NZ), jnp.float32),
            pltpu.VMEM((_NCHUNK, _CH), jnp.float32),
            pltpu.SemaphoreType.DMA,
            pltpu.SemaphoreType.DMA,
            pltpu.SemaphoreType.DMA,
            pltpu.SemaphoreType.DMA,
        ],
    )
    def elog(z_h, row_h, col_h, out_h, ridx_a, cidx_a,
             zr0, zr1, zc0, zc1, tr0, tr1, tc0, tc1, dot_a,
             semr0, semr1, semc0, semc1):
        c = lax.axis_index("c")
        s = lax.axis_index("s")
        wid = s * _NC + c
        zr = (zr0, zr1)
        zc = (zc0, zc1)
        tr = (tr0, tr1)
        tc = (tc0, tc1)
        semr = (semr0, semr1)
        semc = (semc0, semc1)

        pltpu.sync_copy(row_h.at[wid], ridx_a)
        pltpu.sync_copy(col_h.at[wid], cidx_a)

        def issue(j, b):
            pltpu.async_copy(z_h.at[0].at[ridx_a.at[j]], zr[b], semr[b])
            pltpu.async_copy(z_h.at[1].at[ridx_a.at[j]], tr[b], semr[b])
            pltpu.async_copy(z_h.at[0].at[cidx_a.at[j]], zc[b], semc[b])
            pltpu.async_copy(z_h.at[1].at[cidx_a.at[j]], tc[b], semc[b])

        def wait(b):
            pltpu.make_async_copy(z_h.at[0].at[pl.ds(0, _CH)], zr[b],
                                  semr[b]).wait()
            pltpu.make_async_copy(z_h.at[0].at[pl.ds(0, _CH)], tr[b],
                                  semr[b]).wait()
            pltpu.make_async_copy(z_h.at[0].at[pl.ds(0, _CH)], zc[b],
                                  semc[b]).wait()
            pltpu.make_async_copy(z_h.at[0].at[pl.ds(0, _CH)], tc[b],
                                  semc[b]).wait()

        def dots(j, b):
            def grp(g, cc):
                eidx = g * 16 + lax.iota(jnp.int32, 16)
                acc = jnp.zeros((16,), jnp.float32)
                for k in range(_NZ):
                    kidx = jnp.full((16,), k, jnp.int32)
                    ar = (plsc.load_gather(zr[b], [eidx, kidx])
                          + plsc.load_gather(tr[b], [eidx, kidx]))
                    ac = (plsc.load_gather(zc[b], [eidx, kidx])
                          + plsc.load_gather(tc[b], [eidx, kidx]))
                    acc = acc + ar * ac
                dot_a[j, pl.ds(g * 16, 16)] = acc
                return cc

            lax.fori_loop(0, _CH // 16, grp, 0)

        issue(0, 0)

        def pair(m, carry):
            j0 = 2 * m
            j1 = 2 * m + 1
            wait(0)
            issue(j1, 1)
            dots(j0, 0)
            wait(1)

            @pl.when(j1 + 1 < _NCHUNK)
            def _next():
                issue(j1 + 1, 0)

            dots(j1, 1)
            return carry

        lax.fori_loop(0, _NCHUNK // 2, pair, 0)
        wait(0)
        dots(_NCHUNK - 1, 0)
        pltpu.sync_copy(dot_a, out_h.at[wid])

    return elog


_BM = 1024
_G = (_N + _BM - 1) // _BM


def _dec_body(zr_ref, zc_ref, a_ref, s_ref, acc_ref):
    i = pl.program_id(0)
    j = pl.program_id(1)

    @pl.when((i == 0) & (j == 0))
    def _init():
        acc_ref[0] = 0.0

    a = lax.dot_general(zr_ref[0] + zr_ref[1], zc_ref[0] + zc_ref[1],
                        (((1,), (1,)), ((), ())),
                        preferred_element_type=jnp.float32)
    a_ref[...] = a
    sp = jnp.log1p(jnp.exp(-jnp.abs(a))) + jnp.maximum(a, 0.0)
    acc_ref[0] += jnp.sum(sp)

    @pl.when((i == _G - 1) & (j == _G - 1))
    def _fin():
        s_ref[0, 0] = acc_ref[0]


_NPAD = _G * _BM * _G * _BM - _N * _N


def _decoder(zpad):
    return pl.pallas_call(
        _dec_body,
        grid=(_G, _G),
        in_specs=[pl.BlockSpec((2, _BM, _NZ), lambda i, j: (0, i, 0)),
                  pl.BlockSpec((2, _BM, _NZ), lambda i, j: (0, j, 0))],
        out_specs=[pl.BlockSpec((_BM, _BM), lambda i, j: (i, j)),
                   pl.BlockSpec((1, 1), lambda i, j: (0, 0),
                                memory_space=pltpu.SMEM)],
        out_shape=[jax.ShapeDtypeStruct((_N, _N), jnp.float32),
                   jax.ShapeDtypeStruct((1, 1), jnp.float32)],
        scratch_shapes=[pltpu.SMEM((1,), jnp.float32)],
    )(zpad, zpad)


def _loss_body(s_ref, x_ref, o_ref):
    x = x_ref[...]
    lp = jnp.log1p(jnp.exp(-jnp.abs(x)))
    corr = _POS_W * (lp + jnp.maximum(-x, 0.0)) - (lp + jnp.maximum(x, 0.0))
    zero = jnp.float32(0.0)
    sp0 = jnp.log1p(jnp.exp(-jnp.abs(zero))) + jnp.maximum(zero, 0.0)
    total = s_ref[0, 0] - jnp.float32(_NPAD) * sp0 + jnp.sum(corr)
    o_ref[0, 0] = _NORM * (total / float(_N * _N))


def _loss_finish(s, elog2d):
    return pl.pallas_call(
        _loss_body,
        in_specs=[pl.BlockSpec(memory_space=pltpu.SMEM),
                  pl.BlockSpec((_E // 128, 128), lambda: (0, 0))],
        out_specs=pl.BlockSpec(memory_space=pltpu.SMEM),
        out_shape=jax.ShapeDtypeStruct((1, 1), jnp.float32),
    )(s, elog2d)


def kernel(X, W1, W2, adj_vals, edge_index):
    row = edge_index[0].reshape(_NW, _NCHUNK, _CH)
    col = edge_index[1].reshape(_NW, _NCHUNK, _CH)
    val = adj_vals.reshape(_NW, _NCHUNK, _CH)

    xw1 = _matmul(X, W1)
    p1 = _make_spmm(_NH)(xw1, row, col, val)
    hw2 = _layer2(p1, W2)
    p2 = _make_spmm(_NZ, _G * _BM)(hw2, row, col, val)
    elog = _make_edge_logits()(p2, row, col)
    a, s = _decoder(p2)
    loss2d = _loss_finish(s, elog.reshape(_E // 128, 128))
    return (a, loss2d[0, 0])


# speedup vs baseline: 11.6058x; 1.6091x over previous
import functools

import jax
import jax.numpy as jnp
from jax import lax
from jax.experimental import pallas as pl
from jax.experimental.pallas import tpu as pltpu
from jax.experimental.pallas import tpu_sc as plsc

_N = 10000
_E = 320000
_NX = 128
_NH = 64
_NZ = 16

_POS_W = float(_N * _N - _E) / float(_E)
_NORM = float(_N * _N) / float((_N * _N - _E) * 2)

_NC = 2
_NS = 16
_NW = _NC * _NS
_EW = _E // _NW
_CH = 80
_NCHUNK = _EW // _CH
_RB = 624
_RTAIL = _N - _NS * _RB


def _mm_body(x_ref, w_ref, o_ref):
    o_ref[...] = jnp.dot(x_ref[...], w_ref[...],
                         preferred_element_type=jnp.float32)


def _matmul(x, w, rb=400):
    n, k = x.shape
    m = w.shape[1]
    return pl.pallas_call(
        _mm_body,
        grid=(n // rb,),
        in_specs=[pl.BlockSpec((rb, k), lambda i: (i, 0)),
                  pl.BlockSpec((k, m), lambda i: (0, 0))],
        out_specs=pl.BlockSpec((rb, m), lambda i: (i, 0)),
        out_shape=jax.ShapeDtypeStruct((n, m), jnp.float32),
    )(x, w)


def _layer2_body(p_ref, w_ref, o_ref):
    h = jax.nn.relu(p_ref[0] + p_ref[1])
    o_ref[...] = jnp.dot(h, w_ref[...], preferred_element_type=jnp.float32)


def _layer2(p, w, rb=400):
    _, n, k = p.shape
    m = w.shape[1]
    return pl.pallas_call(
        _layer2_body,
        grid=(n // rb,),
        in_specs=[pl.BlockSpec((2, rb, k), lambda i: (0, i, 0)),
                  pl.BlockSpec((k, m), lambda i: (0, 0))],
        out_specs=pl.BlockSpec((rb, m), lambda i: (i, 0)),
        out_shape=jax.ShapeDtypeStruct((n, m), jnp.float32),
    )(p, w)


def _addp_body(p_ref, o_ref):
    o_ref[...] = p_ref[0] + p_ref[1]


def _add_partials(p, rb=400):
    _, n, k = p.shape
    return pl.pallas_call(
        _addp_body,
        grid=(n // rb,),
        in_specs=[pl.BlockSpec((2, rb, k), lambda i: (0, i, 0))],
        out_specs=pl.BlockSpec((rb, k), lambda i: (i, 0)),
        out_shape=jax.ShapeDtypeStruct((n, k), jnp.float32),
    )(p)


def _make_spmm(d, n_out=_N):
    mesh = plsc.VectorSubcoreMesh(core_axis_name="c", subcore_axis_name="s")

    @functools.partial(
        pl.kernel,
        out_type=jax.ShapeDtypeStruct((2, n_out, d), jnp.float32),
        mesh=mesh,
        compiler_params=pltpu.CompilerParams(use_tc_tiling_on_sc=False),
        scratch_types=[
            pltpu.VMEM((_NCHUNK, _CH), jnp.int32),
            pltpu.VMEM((_NCHUNK, _CH), jnp.int32),
            pltpu.VMEM((_NCHUNK, _CH), jnp.float32),
            pltpu.VMEM((_CH, d), jnp.float32),
            pltpu.VMEM((_CH, d), jnp.float32),
            pltpu.VMEM((_CH, d), jnp.float32),
            pltpu.VMEM((_CH, d), jnp.float32),
            pltpu.VMEM((_CH, d), jnp.float32),
            pltpu.VMEM((_CH, d), jnp.float32),
            pltpu.VMEM((_CH, d), jnp.float32),
            pltpu.VMEM_SHARED((_N, d), jnp.float32),
            pltpu.SemaphoreType.DMA,
            pltpu.SemaphoreType.DMA,
            pltpu.SemaphoreType.DMA,
            pltpu.SemaphoreType.DMA,
            pltpu.SemaphoreType.DMA,
            pltpu.SemaphoreType.DMA,
        ],
    )
    def spmm(table_h, row_h, col_h, val_h, out_h,
             ridx_a, cidx_a, val_a, g0, g1, g2, g3, s0, s1, zer_v, acc_sh,
             semg0, semg1, semg2, semg3, sems0, sems1):
        c = lax.axis_index("c")
        s = lax.axis_index("s")
        wid = s * _NC + c
        gbuf = (g0, g1, g2, g3)
        sbuf = (s0, s1)
        semg = (semg0, semg1, semg2, semg3)
        sems = (sems0, sems1)

        def zrow(r, carry):
            for kk in range(d // 16):
                zer_v[r, pl.ds(kk * 16, 16)] = jnp.zeros((16,), jnp.float32)
            return carry

        lax.fori_loop(0, _CH, zrow, 0)
        for q in range(_RB // _CH):
            pltpu.sync_copy(zer_v,
                            acc_sh.at[pl.ds(s * _RB + q * _CH, _CH)])
        pltpu.sync_copy(zer_v.at[pl.ds(0, _RB - (_RB // _CH) * _CH)],
                        acc_sh.at[pl.ds(s * _RB + (_RB // _CH) * _CH,
                                        _RB - (_RB // _CH) * _CH)])

        @pl.when(s == _NS - 1)
        def _ztail():
            pltpu.sync_copy(zer_v.at[pl.ds(0, _RTAIL)],
                            acc_sh.at[pl.ds(_NS * _RB, _RTAIL)])

        pltpu.sync_copy(row_h.at[wid], ridx_a)
        pltpu.sync_copy(col_h.at[wid], cidx_a)
        pltpu.sync_copy(val_h.at[wid], val_a)
        plsc.subcore_barrier()

        def issue_gather(j, b):
            pltpu.async_copy(table_h.at[cidx_a.at[j]], gbuf[b], semg[b])

        def wait_gather(b):
            pltpu.make_async_copy(table_h.at[pl.ds(0, _CH)], gbuf[b],
                                  semg[b]).wait()

        def issue_scatter(j, b):
            pltpu.async_copy(sbuf[b], acc_sh.at[ridx_a.at[j]], sems[b],
                             add=True)

        def wait_scatter(b):
            pltpu.make_async_copy(table_h.at[pl.ds(0, _CH)], sbuf[b],
                                  sems[b]).wait()

        def scale(j, gb, sb):
            def grp(g, cc):
                vals16 = val_a[j, pl.ds(g * 16, 16)]
                for t in range(16):
                    v = vals16[t]
                    e = g * 16 + t
                    for kk in range(d // 16):
                        sl = pl.ds(kk * 16, 16)
                        sbuf[sb][e, sl] = gbuf[gb][e, sl] * v
                return cc

            lax.fori_loop(0, _CH // 16, grp, 0)

        for b in range(4):
            issue_gather(b, b)

        def step(j, gb, sb, first, static_j):
            wait_gather(gb)
            if not first:
                wait_scatter(sb)
            scale(j, gb, sb)
            issue_scatter(j, sb)
            if static_j:
                if j + 4 < _NCHUNK:
                    issue_gather(j + 4, gb)
            else:
                @pl.when(j + 4 < _NCHUNK)
                def _next():
                    issue_gather(j + 4, gb)

        step(0, 0, 0, True, True)
        step(1, 1, 1, True, True)
        step(2, 2, 0, False, True)
        step(3, 3, 1, False, True)

        def quad(m, carry):
            j = 4 * m + 4
            step(j, 0, 0, False, False)
            step(j + 1, 1, 1, False, False)
            step(j + 2, 2, 0, False, False)
            step(j + 3, 3, 1, False, False)
            return carry

        lax.fori_loop(0, (_NCHUNK - 5) // 4, quad, 0)
        step(_NCHUNK - 1, 0, 0, False, True)
        wait_scatter(1)
        wait_scatter(0)
        plsc.subcore_barrier()
        pltpu.sync_copy(acc_sh.at[pl.ds(s * _RB, _RB)],
                        out_h.at[c, pl.ds(s * _RB, _RB)])

        @pl.when(s == _NS - 1)
        def _wtail():
            pltpu.sync_copy(acc_sh.at[pl.ds(_NS * _RB, _RTAIL)],
                            out_h.at[c, pl.ds(_NS * _RB, _RTAIL)])

        if n_out > _N:
            @pl.when(s == 0)
            def _ptail():
                for q in range((n_out - _N) // _CH):
                    pltpu.sync_copy(zer_v,
                                    out_h.at[c, pl.ds(_N + q * _CH, _CH)])

    return spmm


def _make_edge_logits():
    mesh = plsc.VectorSubcoreMesh(core_axis_name="c", subcore_axis_name="s")

    @functools.partial(
        pl.kernel,
        out_type=jax.ShapeDtypeStruct((_NW, _NCHUNK, _CH), jnp.float32),
        mesh=mesh,
        compiler_params=pltpu.CompilerParams(use_tc_tiling_on_sc=False,
                                             needs_layout_passes=False),
        scratch_types=[
            pltpu.VMEM((_NCHUNK, _CH), jnp.int32),
            pltpu.VMEM((_NCHUNK, _CH), jnp.int32),
            pltpu.VMEM((_CH, _NZ), jnp.float32),
            pltpu.VMEM((_CH, _NZ), jnp.float32),
            pltpu.VMEM((_CH, _NZ), jnp.float32),
            pltpu.VMEM((_CH, _NZ), jnp.float32),
            pltpu.VMEM((_CH, _NZ), jnp.float32),
            pltpu.VMEM((_CH, _NZ), jnp.float32),
            pltpu.VMEM((_CH, _NZ), jnp.float32),
            pltpu.VMEM((_CH, _NZ), jnp.float32),
            pltpu.VMEM((_NCHUNK, _CH), jnp.float32),
            pltpu.SemaphoreType.DMA,
            pltpu.SemaphoreType.DMA,
            pltpu.SemaphoreType.DMA,
            pltpu.SemaphoreType.DMA,
        ],
    )
    def elog(z_h, row_h, col_h, out_h, ridx_a, cidx_a,
             zr0, zr1, zc0, zc1, tr0, tr1, tc0, tc1, dot_a,
             semr0, semr1, semc0, semc1):
        c = lax.axis_index("c")
        s = lax.axis_index("s")
        wid = s * _NC + c
        zr = (zr0, zr1)
        zc = (zc0, zc1)
        tr = (tr0, tr1)
        tc = (tc0, tc1)
        semr = (semr0, semr1)
        semc = (semc0, semc1)

        pltpu.sync_copy(row_h.at[wid], ridx_a)
        pltpu.sync_copy(col_h.at[wid], cidx_a)

        def issue(j, b):
            pltpu.async_copy(z_h.at[0].at[ridx_a.at[j]], zr[b], semr[b])
            pltpu.async_copy(z_h.at[1].at[ridx_a.at[j]], tr[b], semr[b])
            pltpu.async_copy(z_h.at[0].at[cidx_a.at[j]], zc[b], semc[b])
            pltpu.async_copy(z_h.at[1].at[cidx_a.at[j]], tc[b], semc[b])

        def wait(b):
            pltpu.make_async_copy(z_h.at[0].at[pl.ds(0, _CH)], zr[b],
                                  semr[b]).wait()
            pltpu.make_async_copy(z_h.at[0].at[pl.ds(0, _CH)], tr[b],
                                  semr[b]).wait()
            pltpu.make_async_copy(z_h.at[0].at[pl.ds(0, _CH)], zc[b],
                                  semc[b]).wait()
            pltpu.make_async_copy(z_h.at[0].at[pl.ds(0, _CH)], tc[b],
                                  semc[b]).wait()

        def dots(j, b):
            def grp(g, cc):
                eidx = g * 16 + lax.iota(jnp.int32, 16)
                acc = jnp.zeros((16,), jnp.float32)
                for k in range(_NZ):
                    kidx = jnp.full((16,), k, jnp.int32)
                    ar = (plsc.load_gather(zr[b], [eidx, kidx])
                          + plsc.load_gather(tr[b], [eidx, kidx]))
                    ac = (plsc.load_gather(zc[b], [eidx, kidx])
                          + plsc.load_gather(tc[b], [eidx, kidx]))
                    acc = acc + ar * ac
                dot_a[j, pl.ds(g * 16, 16)] = acc
                return cc

            lax.fori_loop(0, _CH // 16, grp, 0)

        issue(0, 0)

        def pair(m, carry):
            j0 = 2 * m
            j1 = 2 * m + 1
            wait(0)
            issue(j1, 1)
            dots(j0, 0)
            wait(1)

            @pl.when(j1 + 1 < _NCHUNK)
            def _next():
                issue(j1 + 1, 0)

            dots(j1, 1)
            return carry

        lax.fori_loop(0, _NCHUNK // 2, pair, 0)
        wait(0)
        dots(_NCHUNK - 1, 0)
        pltpu.sync_copy(dot_a, out_h.at[wid])

    return elog


_BM = 1024
_G = (_N + _BM - 1) // _BM


def _dec_body(zr_ref, zc_ref, a_ref, s_ref, acc_ref):
    i = pl.program_id(0)
    j = pl.program_id(1)

    @pl.when((i == 0) & (j == 0))
    def _init():
        acc_ref[0] = 0.0

    a = lax.dot_general(zr_ref[0] + zr_ref[1], zc_ref[0] + zc_ref[1],
                        (((1,), (1,)), ((), ())),
                        preferred_element_type=jnp.float32)
    a_ref[...] = a
    sp = jnp.log1p(jnp.exp(-jnp.abs(a))) + jnp.maximum(a, 0.0)
    acc_ref[0] += jnp.sum(sp)

    @pl.when((i == _G - 1) & (j == _G - 1))
    def _fin():
        s_ref[0, 0] = acc_ref[0]


_NPAD = _G * _BM * _G * _BM - _N * _N


def _decoder(zpad):
    return pl.pallas_call(
        _dec_body,
        grid=(_G, _G),
        in_specs=[pl.BlockSpec((2, _BM, _NZ), lambda i, j: (0, i, 0)),
                  pl.BlockSpec((2, _BM, _NZ), lambda i, j: (0, j, 0))],
        out_specs=[pl.BlockSpec((_BM, _BM), lambda i, j: (i, j)),
                   pl.BlockSpec((1, 1), lambda i, j: (0, 0),
                                memory_space=pltpu.SMEM)],
        out_shape=[jax.ShapeDtypeStruct((_N, _N), jnp.float32),
                   jax.ShapeDtypeStruct((1, 1), jnp.float32)],
        scratch_shapes=[pltpu.SMEM((1,), jnp.float32)],
    )(zpad, zpad)


def _loss_body(s_ref, x_ref, o_ref):
    x = x_ref[...]
    lp = jnp.log1p(jnp.exp(-jnp.abs(x)))
    corr = _POS_W * (lp + jnp.maximum(-x, 0.0)) - (lp + jnp.maximum(x, 0.0))
    zero = jnp.float32(0.0)
    sp0 = jnp.log1p(jnp.exp(-jnp.abs(zero))) + jnp.maximum(zero, 0.0)
    total = s_ref[0, 0] - jnp.float32(_NPAD) * sp0 + jnp.sum(corr)
    o_ref[0, 0] = _NORM * (total / float(_N * _N))


def _loss_finish(s, elog2d):
    return pl.pallas_call(
        _loss_body,
        in_specs=[pl.BlockSpec(memory_space=pltpu.SMEM),
                  pl.BlockSpec((_E // 128, 128), lambda: (0, 0))],
        out_specs=pl.BlockSpec(memory_space=pltpu.SMEM),
        out_shape=jax.ShapeDtypeStruct((1, 1), jnp.float32),
    )(s, elog2d)


def kernel(X, W1, W2, adj_vals, edge_index):
    row = edge_index[0].reshape(_NW, _NCHUNK, _CH)
    col = edge_index[1].reshape(_NW, _NCHUNK, _CH)
    val = adj_vals.reshape(_NW, _NCHUNK, _CH)

    xw1 = _matmul(X, W1)
    p1 = _make_spmm(_NH)(xw1, row, col, val)
    hw2 = _layer2(p1, W2)
    p2 = _make_spmm(_NZ, _G * _BM)(hw2, row, col, val)
    elog = _make_edge_logits()(p2, row, col)
    a, s = _decoder(p2)
    loss2d = _loss_finish(s, elog.reshape(_E // 128, 128))
    return (a, loss2d[0, 0])
